# baseline (speedup 1.0000x reference)
import functools

import jax
import jax.numpy as jnp
from jax import lax
from jax.experimental import pallas as pl
from jax.experimental.pallas import tpu as pltpu
from jax.experimental.pallas import tpu_sc as plsc

_LANES = 16
_C = 64
_NB = 4
_NW = 32
_NSUB = 16
_CZ = 128


def _spmm_sc(n_pad, d, ch0, ch1):
    rows_per_sub = n_pad // _NSUB
    kcopies = rows_per_sub // _CZ
    gch = 40
    mesh = plsc.VectorSubcoreMesh(core_axis_name="c", subcore_axis_name="s")

    @functools.partial(
        pl.kernel,
        mesh=mesh,
        out_type=jax.ShapeDtypeStruct((2, n_pad, d), jnp.float32),
        scratch_types=[
            pltpu.VMEM((gch, _C), jnp.int32),
            pltpu.VMEM((gch, _C), jnp.int32),
            pltpu.VMEM((gch, _C), jnp.float32),
            pltpu.VMEM((_NB * _C, d), jnp.float32),
            pltpu.VMEM_SHARED((n_pad, d), jnp.float32),
        ] + [pltpu.SemaphoreType.DMA] * _NB,
    )
    def spmm(sup_hbm, src_hbm, dst_hbm, w_hbm, out_hbm,
             src_v, dst_v, w_v, ring_v, acc_sh, *sems):
        cid = lax.axis_index("c")
        sid = lax.axis_index("s")

        zvec = jnp.zeros((_LANES,), jnp.float32)

        def zrow(r, _):
            for j in range(d // _LANES):
                ring_v[r, pl.ds(j * _LANES, _LANES)] = zvec
            return 0

        with jax.named_scope("zinit"):
            lax.fori_loop(0, _CZ, zrow, 0)
            for k in range(kcopies):
                pltpu.sync_copy(
                    ring_v.at[pl.ds(0, _CZ)],
                    acc_sh.at[pl.ds(sid * rows_per_sub + k * _CZ, _CZ)])
            plsc.subcore_barrier()

        base_ch = jnp.where(cid == 0, sid * ch0, _NSUB * ch0 + sid * ch1)
        n_groups = jnp.where(cid == 0, ch0 // gch, ch1 // gch)

        def group(gidx, _):
            gbase = base_ch + gidx * gch
            pltpu.sync_copy(src_hbm.at[pl.ds(gbase, gch)], src_v)
            pltpu.sync_copy(dst_hbm.at[pl.ds(gbase, gch)], dst_v)
            pltpu.sync_copy(w_hbm.at[pl.ds(gbase, gch)], w_v)
            for b in range(_NB - 1):
                pltpu.async_copy(sup_hbm.at[src_v.at[b]],
                                 ring_v.at[pl.ds(b * _C, _C)], sems[b])

            def quad(q, _):
                for b in range(_NB):
                    c = _NB * q + b
                    buf = ring_v.at[pl.ds(b * _C, _C)]
                    pltpu.make_async_copy(sup_hbm.at[pl.ds(0, _C)],
                                          buf, sems[b]).wait()

                    @pl.when(c + _NB - 1 < gch)
                    def _():
                        nb = (b + _NB - 1) % _NB
                        pltpu.async_copy(
                            sup_hbm.at[src_v.at[c + _NB - 1]],
                            ring_v.at[pl.ds(nb * _C, _C)], sems[nb])

                    def grp(g, _):
                        wvec = w_v[c, pl.ds(g * _LANES, _LANES)]
                        for l in range(_LANES):
                            wv = lax.broadcast_in_dim(wvec[l], (_LANES,), ())
                            ei = b * _C + g * _LANES + l
                            for j in range(d // _LANES):
                                sl = pl.ds(j * _LANES, _LANES)
                                ring_v[ei, sl] = ring_v[ei, sl] * wv
                        return 0

                    lax.fori_loop(0, _C // _LANES, grp, 0)
                    pltpu.sync_copy(buf, acc_sh.at[dst_v.at[c]], add=True)
                return 0

            lax.fori_loop(0, gch // _NB, quad, 0)
            return 0

        with jax.named_scope("edges"):
            lax.fori_loop(0, n_groups, group, 0)
            plsc.subcore_barrier()

        with jax.named_scope("copyout"):
            for k in range(kcopies):
                off = sid * rows_per_sub + k * _CZ
                pltpu.sync_copy(acc_sh.at[pl.ds(off, _CZ)],
                                out_hbm.at[cid, pl.ds(off, _CZ)])

    return spmm


def _mm_body(x_ref, w_ref, o_ref):
    o_ref[...] = jnp.dot(x_ref[...], w_ref[...],
                         preferred_element_type=jnp.float32)


def _matmul_tc(x, w, blk=512):
    m, k = x.shape
    n = w.shape[1]
    return pl.pallas_call(
        _mm_body,
        grid=(m // blk,),
        in_specs=[pl.BlockSpec((blk, k), lambda i: (i, 0)),
                  pl.BlockSpec((k, n), lambda i: (0, 0))],
        out_specs=pl.BlockSpec((blk, n), lambda i: (i, 0)),
        out_shape=jax.ShapeDtypeStruct((m, n), jnp.float32),
    )(x, w)


def _relu_body(p_ref, b_ref, o_ref):
    o_ref[...] = jnp.maximum(p_ref[0] + p_ref[1] + b_ref[...], 0.0)


def _relu_tc(p, b, blk=512):
    _, m, k = p.shape
    return pl.pallas_call(
        _relu_body,
        grid=(m // blk,),
        in_specs=[pl.BlockSpec((2, blk, k), lambda i: (0, i, 0)),
                  pl.BlockSpec((1, k), lambda i: (0, 0))],
        out_specs=pl.BlockSpec((blk, k), lambda i: (i, 0)),
        out_shape=jax.ShapeDtypeStruct((m, k), jnp.float32),
    )(p, b)


def _lsm_body(p_ref, w_ref, b_ref, o_ref):
    s = p_ref[0] + p_ref[1]
    t = jnp.dot(s, w_ref[...], preferred_element_type=jnp.float32) + b_ref[...]
    m = jnp.max(t, axis=-1, keepdims=True)
    e = jnp.exp(t - m)
    lse = jnp.log(jnp.sum(e, axis=-1, keepdims=True))
    o_ref[...] = t - m - lse


def _lsm_tc(p, w, b, blk=512):
    _, m, k = p.shape
    n = w.shape[1]
    return pl.pallas_call(
        _lsm_body,
        grid=(m // blk,),
        in_specs=[pl.BlockSpec((2, blk, k), lambda i: (0, i, 0)),
                  pl.BlockSpec((k, n), lambda i: (0, 0)),
                  pl.BlockSpec((1, n), lambda i: (0, 0))],
        out_specs=pl.BlockSpec((blk, n), lambda i: (i, 0)),
        out_shape=jax.ShapeDtypeStruct((m, n), jnp.float32),
    )(p, w, b)


def kernel(x, edge_index, edge_weight, W1, b1, W2, b2):
    n, nfeat = x.shape
    e = edge_weight.shape[0]
    nhid = W1.shape[1]
    nclass = W2.shape[1]

    n_pad = -(-n // (_NSUB * _CZ)) * (_NSUB * _CZ)
    pair_ch = -(-e // (_NSUB * _C * 80)) * 80
    e_pad = _NSUB * pair_ch * _C
    tot_ch = e_pad // _C
    ch0 = pair_ch
    ch1 = 0

    src = jnp.pad(edge_index[0].astype(jnp.int32), (0, e_pad - e))
    dst = jnp.pad(edge_index[1].astype(jnp.int32), (0, e_pad - e))
    ew = jnp.pad(edge_weight.astype(jnp.float32), (0, e_pad - e))
    src2d = src.reshape(tot_ch, _C)
    dst2d = dst.reshape(tot_ch, _C)
    ew2d = ew.reshape(tot_ch, _C)

    x_pad = jnp.pad(x, ((0, n_pad - n), (0, 0)))
    sup1 = _matmul_tc(x_pad, W1)

    spmm = _spmm_sc(n_pad, nhid, ch0, ch1)
    p1 = spmm(sup1, src2d, dst2d, ew2d)

    h = _relu_tc(p1, b1.reshape(1, nhid))

    p2 = spmm(h, src2d, dst2d, ew2d)

    w2p = jnp.pad(W2, ((0, 0), (0, 128 - nclass)))
    b2p = jnp.concatenate(
        [b2, jnp.full((128 - nclass,), -1e30, jnp.float32)]).reshape(1, 128)
    out = _lsm_tc(p2, w2p, b2p)
    return out[:n, :nclass]


# speedup vs baseline: 1.3726x; 1.3726x over previous
import functools

import jax
import jax.numpy as jnp
from jax import lax
from jax.experimental import pallas as pl
from jax.experimental.pallas import tpu as pltpu
from jax.experimental.pallas import tpu_sc as plsc

_LANES = 16
_C = 64
_NB = 4
_NW = 32
_NSUB = 16
_CZ = 128


def _spmm_sc(n_pad, d, ch0, ch1):
    rows_per_sub = n_pad // _NSUB
    kcopies = rows_per_sub // _CZ
    gch = 40
    mesh = plsc.VectorSubcoreMesh(core_axis_name="c", subcore_axis_name="s")

    @functools.partial(
        pl.kernel,
        mesh=mesh,
        out_type=jax.ShapeDtypeStruct((2, n_pad, d), jnp.float32),
        scratch_types=[
            pltpu.VMEM((gch, _C), jnp.int32),
            pltpu.VMEM((gch, _C), jnp.int32),
            pltpu.VMEM((gch, _C), jnp.float32),
            pltpu.VMEM((_NB * _C, d), jnp.float32),
            pltpu.VMEM_SHARED((n_pad, d), jnp.float32),
        ] + [pltpu.SemaphoreType.DMA] * _NB,
    )
    def spmm(sup_hbm, src_hbm, dst_hbm, w_hbm, out_hbm,
             src_v, dst_v, w_v, ring_v, acc_sh, *sems):
        cid = lax.axis_index("c")
        sid = lax.axis_index("s")

        zvec = jnp.zeros((_LANES,), jnp.float32)

        def zrow(r, _):
            for j in range(d // _LANES):
                ring_v[r, pl.ds(j * _LANES, _LANES)] = zvec
            return 0

        with jax.named_scope("zinit"):
            lax.fori_loop(0, _CZ, zrow, 0)
            for k in range(kcopies):
                pltpu.sync_copy(
                    ring_v.at[pl.ds(0, _CZ)],
                    acc_sh.at[pl.ds(sid * rows_per_sub + k * _CZ, _CZ)])
            plsc.subcore_barrier()

        base_ch = jnp.where(cid == 0, sid * ch0, _NSUB * ch0 + sid * ch1)
        n_groups = jnp.where(cid == 0, ch0 // gch, ch1 // gch)

        def group(gidx, _):
            gbase = base_ch + gidx * gch
            pltpu.sync_copy(src_hbm.at[pl.ds(gbase, gch)], src_v)
            pltpu.sync_copy(dst_hbm.at[pl.ds(gbase, gch)], dst_v)
            pltpu.sync_copy(w_hbm.at[pl.ds(gbase, gch)], w_v)
            for b in range(_NB - 1):
                pltpu.async_copy(sup_hbm.at[src_v.at[b]],
                                 ring_v.at[pl.ds(b * _C, _C)], sems[b])

            def quad(q, _):
                for b in range(_NB):
                    c = _NB * q + b
                    buf = ring_v.at[pl.ds(b * _C, _C)]
                    pltpu.make_async_copy(sup_hbm.at[pl.ds(0, _C)],
                                          buf, sems[b]).wait()

                    @pl.when(c + _NB - 1 < gch)
                    def _():
                        nb = (b + _NB - 1) % _NB
                        pltpu.async_copy(
                            sup_hbm.at[src_v.at[c + _NB - 1]],
                            ring_v.at[pl.ds(nb * _C, _C)], sems[nb])

                    def grp(g, _):
                        wvec = w_v[c, pl.ds(g * _LANES, _LANES)]
                        for l in range(_LANES):
                            wv = lax.broadcast_in_dim(wvec[l], (_LANES,), ())
                            ei = b * _C + g * _LANES + l
                            for j in range(d // _LANES):
                                sl = pl.ds(j * _LANES, _LANES)
                                ring_v[ei, sl] = ring_v[ei, sl] * wv
                        return 0

                    lax.fori_loop(0, _C // _LANES, grp, 0)
                    pltpu.sync_copy(buf, acc_sh.at[dst_v.at[c]], add=True)
                return 0

            lax.fori_loop(0, gch // _NB, quad, 0)
            return 0

        with jax.named_scope("edges"):
            lax.fori_loop(0, n_groups, group, 0)
            plsc.subcore_barrier()

        with jax.named_scope("copyout"):
            for k in range(kcopies):
                off = sid * rows_per_sub + k * _CZ
                pltpu.sync_copy(acc_sh.at[pl.ds(off, _CZ)],
                                out_hbm.at[cid, pl.ds(off, _CZ)])

    return spmm


def _mm_body(x_ref, w_ref, o_ref):
    o_ref[...] = jnp.dot(x_ref[...], w_ref[...],
                         preferred_element_type=jnp.float32)


def _matmul_tc(x, w, blk=512):
    m, k = x.shape
    n = w.shape[1]
    return pl.pallas_call(
        _mm_body,
        grid=(m // blk,),
        in_specs=[pl.BlockSpec((blk, k), lambda i: (i, 0)),
                  pl.BlockSpec((k, n), lambda i: (0, 0))],
        out_specs=pl.BlockSpec((blk, n), lambda i: (i, 0)),
        out_shape=jax.ShapeDtypeStruct((m, n), jnp.float32),
    )(x, w)


def _relu_body(p_ref, b_ref, o_ref):
    o_ref[...] = jnp.maximum(p_ref[0] + p_ref[1] + b_ref[...], 0.0)


def _relu_tc(p, b, blk=512):
    _, m, k = p.shape
    return pl.pallas_call(
        _relu_body,
        grid=(m // blk,),
        in_specs=[pl.BlockSpec((2, blk, k), lambda i: (0, i, 0)),
                  pl.BlockSpec((1, k), lambda i: (0, 0))],
        out_specs=pl.BlockSpec((blk, k), lambda i: (i, 0)),
        out_shape=jax.ShapeDtypeStruct((m, k), jnp.float32),
    )(p, b)


def _lsm_body(p_ref, w_ref, b_ref, o_ref):
    s = p_ref[0] + p_ref[1]
    t = jnp.dot(s, w_ref[...], preferred_element_type=jnp.float32) + b_ref[...]
    m = jnp.max(t, axis=-1, keepdims=True)
    e = jnp.exp(t - m)
    lse = jnp.log(jnp.sum(e, axis=-1, keepdims=True))
    o_ref[...] = t - m - lse


def _lsm_tc(p, w, b, blk=512):
    _, m, k = p.shape
    n = w.shape[1]
    return pl.pallas_call(
        _lsm_body,
        grid=(m // blk,),
        in_specs=[pl.BlockSpec((2, blk, k), lambda i: (0, i, 0)),
                  pl.BlockSpec((k, n), lambda i: (0, 0)),
                  pl.BlockSpec((1, n), lambda i: (0, 0))],
        out_specs=pl.BlockSpec((blk, n), lambda i: (i, 0)),
        out_shape=jax.ShapeDtypeStruct((m, n), jnp.float32),
    )(p, w, b)


def kernel(x, edge_index, edge_weight, W1, b1, W2, b2):
    n, nfeat = x.shape
    e = edge_weight.shape[0]
    nhid = W1.shape[1]
    nclass = W2.shape[1]

    n_pad = -(-n // (_NSUB * _CZ)) * (_NSUB * _CZ)
    pair_ch = -(-e // (_NSUB * _C * 80)) * 80
    e_pad = _NSUB * pair_ch * _C
    tot_ch = e_pad // _C
    ch0 = pair_ch * 7 // 8
    ch1 = pair_ch - ch0

    src = jnp.pad(edge_index[0].astype(jnp.int32), (0, e_pad - e))
    dst = jnp.pad(edge_index[1].astype(jnp.int32), (0, e_pad - e))
    ew = jnp.pad(edge_weight.astype(jnp.float32), (0, e_pad - e))
    src2d = src.reshape(tot_ch, _C)
    dst2d = dst.reshape(tot_ch, _C)
    ew2d = ew.reshape(tot_ch, _C)

    x_pad = jnp.pad(x, ((0, n_pad - n), (0, 0)))
    sup1 = _matmul_tc(x_pad, W1)

    spmm = _spmm_sc(n_pad, nhid, ch0, ch1)
    p1 = spmm(sup1, src2d, dst2d, ew2d)

    h = _relu_tc(p1, b1.reshape(1, nhid))

    p2 = spmm(h, src2d, dst2d, ew2d)

    w2p = jnp.pad(W2, ((0, 0), (0, 128 - nclass)))
    b2p = jnp.concatenate(
        [b2, jnp.full((128 - nclass,), -1e30, jnp.float32)]).reshape(1, 128)
    out = _lsm_tc(p2, w2p, b2p)
    return out[:n, :nclass]
